# baseline (speedup 1.0000x reference)
import jax
import jax.numpy as jnp
from jax import lax
from jax.experimental import pallas as pl
from jax.experimental.pallas import tpu as pltpu
from jax.experimental.pallas import tpu_sc as plsc

N_VOCAB = 1000000
D = 32
BLK = 128
NC = 2
NS = 16
NW = NC * NS
B = 16384 * 50
B_PER_W = B // NW
CHUNK = 128
NCHUNK = B_PER_W // CHUNK
UNROLL = 4


def _embed_body(ids_hbm, table_hbm, out_hbm, idx_v, g_v, rows_v, out_v, sem_in, sem_out):
    wid = lax.axis_index("s") * NC + lax.axis_index("c")
    base = wid * B_PER_W
    pltpu.sync_copy(ids_hbm.at[pl.ds(base, B_PER_W)], idx_v)

    lane = lax.iota(jnp.int32, 16)
    ev = lane * 2

    def start_gather(j, p):
        for v in range(CHUNK // 16):
            g_v[p][pl.ds(v * 16, 16)] = (
                idx_v[pl.ds(j * CHUNK + v * 16, 16)] >> 2
            )
        pltpu.async_copy(table_hbm.at[g_v[p]], rows_v[p], sem_in[p])

    start_gather(0, 0)
    start_gather(1, 1)

    def pair_body(j0, _):
        for p in range(2):
            j = j0 * 2 + p
            pltpu.make_async_copy(
                table_hbm.at[g_v[p]], rows_v[p], sem_in[p]
            ).wait()
            @pl.when(j >= 2)
            def _():
                pltpu.make_async_copy(
                    out_v[p], out_hbm.at[pl.ds((base + (j - 2) * CHUNK) * D, CHUNK * D)],
                    sem_out[p],
                ).wait()

            def row_body(r0, _):
                qv = (idx_v[pl.ds(j * CHUNK + r0 * 16, 16)] & 3) * D
                for rr in range(16):
                    r = r0 * 16 + rr
                    r_b = jnp.full((16,), r, jnp.int32)
                    ve = plsc.load_gather(rows_v[p], [r_b, qv[rr] + ev])
                    vo = plsc.load_gather(rows_v[p], [r_b, qv[rr] + ev + 1])
                    out_v[p][pl.ds(r * D, D)] = plsc.pack(
                        ve, vo, format=plsc.PackFormat.INTERLEAVED
                    )
                return 0

            lax.fori_loop(0, CHUNK // 16, row_body, 0)

            @pl.when(j + 2 < NCHUNK)
            def _():
                start_gather(j + 2, p)

            pltpu.async_copy(
                out_v[p], out_hbm.at[pl.ds((base + j * CHUNK) * D, CHUNK * D)], sem_out[p]
            )
        return 0

    lax.fori_loop(0, NCHUNK // 2, pair_body, 0)

    for p in range(2):
        j = NCHUNK - 2 + p
        pltpu.make_async_copy(
            out_v[p], out_hbm.at[pl.ds((base + j * CHUNK) * D, CHUNK * D)], sem_out[p]
        ).wait()


@jax.jit
def _embed(ids_flat, table_blk):
    mesh = plsc.VectorSubcoreMesh(
        core_axis_name="c", subcore_axis_name="s", num_cores=NC, num_subcores=NS
    )
    f = pl.kernel(
        _embed_body,
        out_type=jax.ShapeDtypeStruct((B * D,), jnp.bfloat16),
        mesh=mesh,
        scratch_types=[
            pltpu.VMEM((B_PER_W,), jnp.int32),
            [pltpu.VMEM((CHUNK,), jnp.int32) for _ in range(2)],
            [pltpu.VMEM((CHUNK, BLK), jnp.float32) for _ in range(2)],
            [pltpu.VMEM((CHUNK * D,), jnp.bfloat16) for _ in range(2)],
            [pltpu.SemaphoreType.DMA for _ in range(2)],
            [pltpu.SemaphoreType.DMA for _ in range(2)],
        ],
        compiler_params=pltpu.CompilerParams(
            needs_layout_passes=False, use_tc_tiling_on_sc=False
        ),
    )
    return f(ids_flat, table_blk)


def kernel(ids, embedding):
    table_blk = embedding.reshape(N_VOCAB // 4, BLK)
    out = _embed(ids.astype(jnp.int32).reshape(-1), table_blk)
    return out.reshape(ids.shape[0], ids.shape[1], D)


# speedup vs baseline: 1.0633x; 1.0633x over previous
import jax
import jax.numpy as jnp
from jax import lax
from jax.experimental import pallas as pl
from jax.experimental.pallas import tpu as pltpu
from jax.experimental.pallas import tpu_sc as plsc

N_VOCAB = 1000000
D = 32
NC = 2
NS = 16
NW = NC * NS
B = 16384 * 50
B_PER_W = B // NW
CHUNK = 128
NCHUNK = B_PER_W // CHUNK
UNROLL = 4


def _embed_body(ids_hbm, table_hbm, out_hbm, idx_v, rows_v, out_v, sem_in, sem_out):
    wid = lax.axis_index("s") * NC + lax.axis_index("c")
    base = wid * B_PER_W
    pltpu.sync_copy(ids_hbm.at[pl.ds(base, B_PER_W)], idx_v)

    ev = lax.iota(jnp.int32, 16) * 2

    def start_gather(j, p):
        pltpu.async_copy(
            table_hbm.at[idx_v.at[pl.ds(j * CHUNK, CHUNK)]], rows_v[p], sem_in[p]
        )

    start_gather(0, 0)
    start_gather(1, 1)

    def pair_body(j0, _):
        for p in range(2):
            j = j0 * 2 + p
            pltpu.make_async_copy(
                table_hbm.at[idx_v.at[pl.ds(j * CHUNK, CHUNK)]], rows_v[p], sem_in[p]
            ).wait()
            @pl.when(j >= 2)
            def _():
                pltpu.make_async_copy(
                    out_v[p], out_hbm.at[pl.ds((base + (j - 2) * CHUNK) * D, CHUNK * D)],
                    sem_out[p],
                ).wait()

            def row_body(r0, _):
                for rr in range(UNROLL):
                    r = r0 * UNROLL + rr
                    r_b = jnp.full((16,), r, jnp.int32)
                    ve = plsc.load_gather(rows_v[p], [r_b, ev])
                    vo = plsc.load_gather(rows_v[p], [r_b, ev + 1])
                    out_v[p][pl.ds(r * D, D)] = plsc.pack(
                        ve, vo, format=plsc.PackFormat.INTERLEAVED
                    )
                return 0

            lax.fori_loop(0, CHUNK // UNROLL, row_body, 0)

            @pl.when(j + 2 < NCHUNK)
            def _():
                start_gather(j + 2, p)

            pltpu.async_copy(
                out_v[p], out_hbm.at[pl.ds((base + j * CHUNK) * D, CHUNK * D)], sem_out[p]
            )
        return 0

    lax.fori_loop(0, NCHUNK // 2, pair_body, 0)

    for p in range(2):
        j = NCHUNK - 2 + p
        pltpu.make_async_copy(
            out_v[p], out_hbm.at[pl.ds((base + j * CHUNK) * D, CHUNK * D)], sem_out[p]
        ).wait()


@jax.jit
def _embed(ids_flat, table):
    mesh = plsc.VectorSubcoreMesh(
        core_axis_name="c", subcore_axis_name="s", num_cores=NC, num_subcores=NS
    )
    f = pl.kernel(
        _embed_body,
        out_type=jax.ShapeDtypeStruct((B * D,), jnp.bfloat16),
        mesh=mesh,
        scratch_types=[
            pltpu.VMEM((B_PER_W,), jnp.int32),
            [pltpu.VMEM((CHUNK, D), jnp.float32) for _ in range(2)],
            [pltpu.VMEM((CHUNK * D,), jnp.bfloat16) for _ in range(2)],
            [pltpu.SemaphoreType.DMA for _ in range(2)],
            [pltpu.SemaphoreType.DMA for _ in range(2)],
        ],
        compiler_params=pltpu.CompilerParams(
            needs_layout_passes=False, use_tc_tiling_on_sc=False
        ),
    )
    return f(ids_flat, table)


def kernel(ids, embedding):
    out = _embed(ids.astype(jnp.int32).reshape(-1), embedding)
    return out.reshape(ids.shape[0], ids.shape[1], D)
